# baseline (speedup 1.0000x reference)
import functools

import jax
import jax.numpy as jnp
from jax import lax
from jax.experimental import pallas as pl
from jax.experimental.pallas import tpu as pltpu
from jax.experimental.pallas import tpu_sc as plsc

N = 10000
E = 320000
F = 128
De = 16
H = 128
G = 100

NB = 2000
EB = 2560
PW = 32

NWORK = 32
EPW = E // NWORK
CH = 80
NP = 10240
NSTR = NP // 16

_INTERPRET = False


def _silu(v):
    return v * jax.nn.sigmoid(v)


def _wspec(shape):
    nd = len(shape)
    return pl.BlockSpec(shape, lambda *_, **__: (0,) * nd)


def _proj_body(h_ref, ws_ref, wd_ref, ps_ref, pd_ref):
    h = h_ref[...]
    ps_ref[...] = jnp.dot(h, ws_ref[...], preferred_element_type=jnp.float32)
    pd_ref[...] = jnp.dot(h, wd_ref[...], preferred_element_type=jnp.float32)


def _proj(h, ws, wd):
    grid = (N // NB,)
    return pl.pallas_call(
        _proj_body,
        grid=grid,
        in_specs=[pl.BlockSpec((NB, F), lambda i: (i, 0)), _wspec((F, H)), _wspec((F, H))],
        out_specs=[pl.BlockSpec((NB, H), lambda i: (i, 0))] * 2,
        out_shape=[jax.ShapeDtypeStruct((N, H), jnp.float32)] * 2,
        interpret=_INTERPRET,
    )(h, ws, wd)


def _edge_body(first, ga_ref, gb_ref, e_ref, r0_ref, r1_ref, r2_ref,
               we_ref, wd2_ref, be1_ref, we2_ref, be2_ref, wx_ref, bx_ref,
               pay_ref):
    rel_ref = (r0_ref, r1_ref, r2_ref)
    if first:
        e = e_ref[...]
    else:
        e = e_ref[:, 0:De]
    i = pl.program_id(0)
    sl = pl.ds(i * EB, EB)
    r0c = jnp.reshape(rel_ref[0][sl], (EB, 1))
    r1c = jnp.reshape(rel_ref[1][sl], (EB, 1))
    r2c = jnp.reshape(rel_ref[2][sl], (EB, 1))
    d2 = r0c * r0c + r1c * r1c + r2c * r2c
    mpre = (ga_ref[...] + gb_ref[...]
            + jnp.dot(e, we_ref[...], preferred_element_type=jnp.float32)
            + d2 * wd2_ref[...] + be1_ref[...])
    m = _silu(mpre)
    e_new = _silu(jnp.dot(m, we2_ref[...], preferred_element_type=jnp.float32)
                  + be2_ref[...])
    w = jnp.tanh(jnp.dot(e_new, wx_ref[...], preferred_element_type=jnp.float32)
                 + bx_ref[...])
    relw = jnp.concatenate([r0c, r1c, r2c], axis=1) * w
    ones = jnp.ones((EB, 1), jnp.float32)
    zeros = jnp.zeros((EB, PW - De - 4), jnp.float32)
    pay_ref[...] = jnp.concatenate([e_new, relw, ones, zeros], axis=1)


def _edge_dense(first, ga, gb, e, r0, r1, r2, we, wd2, be1, we2, be2, wx, bx):
    grid = (E // EB,)
    e_spec = (pl.BlockSpec((EB, De), lambda i: (i, 0)) if first
              else pl.BlockSpec((EB, PW), lambda i: (i, 0)))
    return pl.pallas_call(
        functools.partial(_edge_body, first),
        grid=grid,
        in_specs=[
            pl.BlockSpec((EB, H), lambda i: (i, 0)),
            pl.BlockSpec((EB, H), lambda i: (i, 0)),
            e_spec,
            _wspec((E,)), _wspec((E,)), _wspec((E,)),
            _wspec((De, H)), _wspec((1, H)), _wspec((1, H)),
            _wspec((H, De)), _wspec((1, De)), _wspec((De, 1)), _wspec((1, 1)),
        ],
        out_specs=pl.BlockSpec((EB, PW), lambda i: (i, 0)),
        out_shape=jax.ShapeDtypeStruct((E, PW), jnp.float32),
        interpret=_INTERPRET,
    )(ga, gb, e, r0, r1, r2, we, wd2, be1, we2, be2, wx, bx)


def _node_body(h_ref, x0_ref, x1_ref, x2_ref, acc_ref, wh1h_ref, wh1a_ref,
               bh1_ref, wh2_ref, bh2_ref, hn_ref, xn0_ref, xn1_ref, xn2_ref):
    acc = acc_ref[0] + acc_ref[1]
    h = h_ref[...]
    inv = 1.0 / jnp.maximum(acc[:, 19:20], 1.0)
    agg = acc[:, 0:16] * inv
    dx = acc[:, 16:19] * inv
    for xr, xnr, c in ((x0_ref, xn0_ref, 0), (x1_ref, xn1_ref, 1),
                       (x2_ref, xn2_ref, 2)):
        xc = jnp.reshape(xr[...], (NB, 1))
        xnr[...] = jnp.reshape(xc + dx[:, c:c + 1], (1, 1, NB))
    hp = _silu(jnp.dot(h, wh1h_ref[...], preferred_element_type=jnp.float32)
               + jnp.dot(agg, wh1a_ref[...], preferred_element_type=jnp.float32)
               + bh1_ref[...])
    hn_ref[...] = h + jnp.dot(hp, wh2_ref[...], preferred_element_type=jnp.float32) + bh2_ref[...]


def _node_update(h, x0, x1, x2, acc2, wh1h, wh1a, bh1, wh2, bh2):
    grid = (N // NB,)
    return pl.pallas_call(
        _node_body,
        grid=grid,
        in_specs=[
            pl.BlockSpec((NB, F), lambda i: (i, 0)),
            pl.BlockSpec((1, 1, NB), lambda i: (i, 0, 0)),
            pl.BlockSpec((1, 1, NB), lambda i: (i, 0, 0)),
            pl.BlockSpec((1, 1, NB), lambda i: (i, 0, 0)),
            pl.BlockSpec((2, NB, PW), lambda i: (0, i, 0)),
            _wspec((F, H)), _wspec((De, H)), _wspec((1, H)),
            _wspec((H, F)), _wspec((1, F)),
        ],
        out_specs=[pl.BlockSpec((NB, F), lambda i: (i, 0)),
                   pl.BlockSpec((1, 1, NB), lambda i: (i, 0, 0)),
                   pl.BlockSpec((1, 1, NB), lambda i: (i, 0, 0)),
                   pl.BlockSpec((1, 1, NB), lambda i: (i, 0, 0))],
        out_shape=[jax.ShapeDtypeStruct((N, F), jnp.float32),
                   jax.ShapeDtypeStruct((N // NB, 1, NB), jnp.float32),
                   jax.ShapeDtypeStruct((N // NB, 1, NB), jnp.float32),
                   jax.ShapeDtypeStruct((N // NB, 1, NB), jnp.float32)],
        interpret=_INTERPRET,
    )(h, x0.reshape(N // NB, 1, NB), x1.reshape(N // NB, 1, NB),
      x2.reshape(N // NB, 1, NB), acc2, wh1h, wh1a, bh1, wh2, bh2)


def _mono_body(h_ref, nf_ref, gid_ref, wm_ref, bm_ref, pm_ref, sums_ref, cnt_ref):
    i = pl.program_id(0)
    h = h_ref[...]
    pm = jnp.dot(h, wm_ref[...], preferred_element_type=jnp.float32) + bm_ref[...]
    mask = nf_ref[:, 0:1] == 1.0
    pm = jnp.where(mask, jnp.abs(pm), pm)
    pm_ref[...] = pm
    gid = gid_ref[0, 0, :]
    oh = (gid[:, None] == jax.lax.broadcasted_iota(jnp.int32, (1, 128), 1)
          ).astype(jnp.float32)
    psum = jnp.dot(oh.T, pm, preferred_element_type=jnp.float32)
    pcnt = jnp.sum(oh, axis=0)[:, None]

    @pl.when(i == 0)
    def _():
        sums_ref[...] = psum
        cnt_ref[...] = pcnt

    @pl.when(i != 0)
    def _():
        sums_ref[...] += psum
        cnt_ref[...] += pcnt


def _mono_readout(h_mon, nfeats, gid3, wm, bm):
    grid = (N // NB,)
    return pl.pallas_call(
        _mono_body,
        grid=grid,
        in_specs=[
            pl.BlockSpec((NB, F), lambda i: (i, 0)),
            pl.BlockSpec((NB, F), lambda i: (i, 0)),
            pl.BlockSpec((1, 1, NB), lambda i: (i, 0, 0)),
            _wspec((F, 1)), _wspec((1, 1)),
        ],
        out_specs=[pl.BlockSpec((NB, 1), lambda i: (i, 0)),
                   _wspec((128, 1)), _wspec((128, 1))],
        out_shape=[jax.ShapeDtypeStruct((N, 1), jnp.float32),
                   jax.ShapeDtypeStruct((128, 1), jnp.float32),
                   jax.ShapeDtypeStruct((128, 1), jnp.float32)],
        interpret=_INTERPRET,
    )(h_mon, nfeats, gid3, wm, bm)


def _fv_body(sums_ref, cnt_ref, fv_ref):
    s = sums_ref[...]
    fv = s / jnp.maximum(cnt_ref[...], 1.0)
    fv_ref[...] = jnp.where(jnp.abs(s) < 0.01, 0.0, fv)


def _fv_finalize(sums, cnt):
    return pl.pallas_call(
        _fv_body,
        in_specs=[_wspec((128, 1)), _wspec((128, 1))],
        out_specs=_wspec((128, 1)),
        out_shape=jax.ShapeDtypeStruct((128, 1), jnp.float32),
        interpret=_INTERPRET,
    )(sums, cnt)


def _final_body(pm_ref, gid_ref, fv_ref, hd_ref, hq_ref, ho_ref,
                wd_ref, bd_ref, wq_ref, bq_ref, wo_ref, bo_ref, out_ref):
    gid = gid_ref[0, 0, :]
    oh = (gid[:, None] == jax.lax.broadcasted_iota(jnp.int32, (1, 128), 1)
          ).astype(jnp.float32)
    pm = pm_ref[...] - jnp.dot(oh, fv_ref[...], preferred_element_type=jnp.float32)
    pd = jnp.dot(hd_ref[...], wd_ref[...], preferred_element_type=jnp.float32) + bd_ref[...]
    pq = jnp.dot(hq_ref[...], wq_ref[...], preferred_element_type=jnp.float32) + bq_ref[...]
    mt = (pq[:, 0:1] + pq[:, 3:4] + pq[:, 5:6]) / 3.0
    c6 = jax.lax.broadcasted_iota(jnp.int32, (1, 6), 1)
    qmask = ((c6 == 0) | (c6 == 3) | (c6 == 5)).astype(jnp.float32)
    pq = pq - mt * qmask
    po = jnp.dot(ho_ref[...], wo_ref[...], preferred_element_type=jnp.float32) + bo_ref[...]
    m0 = (po[:, 0:1] + po[:, 3:4] + po[:, 5:6]) / 3.0
    m1 = (po[:, 6:7] + po[:, 1:2] + po[:, 8:9]) / 3.0
    m2 = (po[:, 9:10] + po[:, 2:3] + po[:, 7:8]) / 3.0
    c10 = jax.lax.broadcasted_iota(jnp.int32, (1, 10), 1)
    g0 = ((c10 == 0) | (c10 == 3) | (c10 == 5)).astype(jnp.float32)
    g1 = ((c10 == 6) | (c10 == 1) | (c10 == 8)).astype(jnp.float32)
    g2 = ((c10 == 9) | (c10 == 2) | (c10 == 7)).astype(jnp.float32)
    po = po - m0 * g0 - m1 * g1 - m2 * g2
    out_ref[...] = jnp.concatenate([pm, pd, pq, po], axis=1)


def _final_readout(pm_raw, gid3, fv, h_dip, h_quad, h_oct, wd, bd, wq, bq, wo, bo):
    grid = (N // NB,)
    return pl.pallas_call(
        _final_body,
        grid=grid,
        in_specs=[
            pl.BlockSpec((NB, 1), lambda i: (i, 0)),
            pl.BlockSpec((1, 1, NB), lambda i: (i, 0, 0)),
            _wspec((128, 1)),
            pl.BlockSpec((NB, F), lambda i: (i, 0)),
            pl.BlockSpec((NB, F), lambda i: (i, 0)),
            pl.BlockSpec((NB, F), lambda i: (i, 0)),
            _wspec((F, 3)), _wspec((1, 3)),
            _wspec((F, 6)), _wspec((1, 6)),
            _wspec((F, 10)), _wspec((1, 10)),
        ],
        out_specs=pl.BlockSpec((NB, 20), lambda i: (i, 0)),
        out_shape=jax.ShapeDtypeStruct((N, 20), jnp.float32),
        interpret=_INTERPRET,
    )(pm_raw, gid3, fv, h_dip, h_quad, h_oct, wd, bd, wq, bq, wo, bo)


NCH = EPW // CH
NPAIR = NCH // 2


def _gather_stage(ps, pd, x0, x1, x2, ei):
    mesh = plsc.VectorSubcoreMesh(core_axis_name="c", subcore_axis_name="s")

    @functools.partial(
        pl.kernel, mesh=mesh,
        out_type=[jax.ShapeDtypeStruct((E, H), jnp.float32),
                  jax.ShapeDtypeStruct((E, H), jnp.float32),
                  jax.ShapeDtypeStruct((E,), jnp.float32),
                  jax.ShapeDtypeStruct((E,), jnp.float32),
                  jax.ShapeDtypeStruct((E,), jnp.float32)],
        scratch_types=[
            pltpu.VMEM((2, NCH, CH), jnp.int32),
            pltpu.VMEM((4, CH, H), jnp.float32),
            pltpu.VMEM((4, CH, H), jnp.float32),
            pltpu.VMEM((4, 6, CH), jnp.float32),
            pltpu.VMEM((4, 3, CH), jnp.float32),
            pltpu.SemaphoreType.DMA,
            pltpu.SemaphoreType.DMA,
            pltpu.SemaphoreType.DMA,
            pltpu.SemaphoreType.DMA,
            pltpu.SemaphoreType.DMA,
            pltpu.SemaphoreType.DMA,
        ],
    )
    def k(ps_hbm, pd_hbm, x0_hbm, x1_hbm, x2_hbm, ei_hbm,
          ga_hbm, gb_hbm, r0_hbm, r1_hbm, r2_hbm,
          eslab, gabuf, gbbuf, xbuf, rbuf, semI, semA, semB, semC, semW, semR):
        wid = lax.axis_index("s") * 2 + lax.axis_index("c")
        xh = (x0_hbm, x1_hbm, x2_hbm)
        rh = (r0_hbm, r1_hbm, r2_hbm)
        pltpu.async_copy(ei_hbm.at[:, wid], eslab, semI).wait()

        def fire_gathers(ci, b):
            src_r = eslab.at[0, ci]
            dst_r = eslab.at[1, ci]
            cpa = pltpu.async_copy(ps_hbm.at[src_r], gabuf.at[b], semA)
            cpb = pltpu.async_copy(pd_hbm.at[dst_r], gbbuf.at[b], semB)
            cps = []
            for d in range(3):
                cps.append(pltpu.async_copy(xh[d].at[src_r], xbuf.at[b, d], semC))
                cps.append(pltpu.async_copy(xh[d].at[dst_r], xbuf.at[b, d + 3], semC))
            return cpa, cpb, cps

        def drain_chunk(ci, b, cpa, cpb, cps):
            base = pl.multiple_of(wid * EPW + ci * CH, 8)
            for cp in cps:
                cp.wait()
            for d in range(3):
                for j in range(CH // 16):
                    sl = pl.ds(j * 16, 16)
                    rbuf[b, d, sl] = xbuf[b, d, sl] - xbuf[b, d + 3, sl]
            wr = [pltpu.async_copy(rbuf.at[b, d], rh[d].at[pl.ds(base, CH)],
                                   semR) for d in range(3)]
            cpa.wait()
            cpb.wait()
            wa = pltpu.async_copy(gabuf.at[b], ga_hbm.at[pl.ds(base, CH)], semW)
            wb = pltpu.async_copy(gbbuf.at[b], gb_hbm.at[pl.ds(base, CH)], semW)
            return wr + [wa, wb]

        def drain_writes():
            for b in range(4):
                for d in range(3):
                    pltpu.make_async_copy(r0_hbm.at[pl.ds(0, CH)],
                                          rbuf.at[b, d], semR).wait()
                pltpu.make_async_copy(ga_hbm.at[pl.ds(0, CH)], gabuf.at[b],
                                      semW).wait()
                pltpu.make_async_copy(gb_hbm.at[pl.ds(0, CH)], gbbuf.at[b],
                                      semW).wait()

        def quad_body(q, carry):
            @pl.when(q > 0)
            def _():
                drain_writes()

            gs = [fire_gathers(q * 4 + j, j) for j in range(4)]
            for j in range(4):
                drain_chunk(q * 4 + j, j, *gs[j])
            return carry

        lax.fori_loop(0, NCH // 4, quad_body, 0)
        drain_writes()
        ct = NCH - 1
        gt = fire_gathers(ct, 0)
        wt = drain_chunk(ct, 0, *gt)
        for w in wt:
            w.wait()

    return k(ps, pd, x0, x1, x2, ei)


def _scatter_stage(pay_flat, dst3, zeros2d):
    mesh = plsc.VectorSubcoreMesh(core_axis_name="c", subcore_axis_name="s")

    @functools.partial(
        pl.kernel, mesh=mesh,
        out_type=jax.ShapeDtypeStruct((2, NP, PW), jnp.float32),
        scratch_types=[
            pltpu.VMEM((NCH, CH), jnp.int32),
            pltpu.VMEM((2, CH, PW), jnp.float32),
            pltpu.VMEM_SHARED((NP, PW), jnp.float32),
            pltpu.SemaphoreType.DMA,
            pltpu.SemaphoreType.DMA,
            pltpu.SemaphoreType.DMA,
        ],
    )
    def k(pay_hbm, dst_hbm, z_hbm, out_hbm, dslab, payv, acc, semD, semP, semS):
        cid = lax.axis_index("c")
        sid = lax.axis_index("s")
        wid = sid * 2 + cid
        stripe = pl.ds(sid * NSTR, NSTR)
        cpd = pltpu.async_copy(dst_hbm.at[wid], dslab, semD)
        pltpu.sync_copy(z_hbm.at[stripe], acc.at[stripe])
        cpd.wait()
        plsc.subcore_barrier()

        def fire_pay(ci, b):
            base = pl.multiple_of(wid * EPW + ci * CH, 8)
            return pltpu.async_copy(pay_hbm.at[pl.ds(base, CH)], payv.at[b],
                                    semP)

        def fire_add(ci, b):
            return pltpu.async_copy(payv.at[b], acc.at[dslab.at[ci]], semS)

        def pair_body(kk, carry):
            c0 = kk * 2
            c1 = c0 + 1
            p0 = fire_pay(c0, 0)
            p1 = fire_pay(c1, 1)
            p0.wait()
            s0 = fire_add(c0, 0)
            p1.wait()
            s1 = fire_add(c1, 1)
            s0.wait()
            s1.wait()
            return carry

        lax.fori_loop(0, NPAIR, pair_body, 0)
        ct = NCH - 1
        pt = fire_pay(ct, 0)
        pt.wait()
        st = fire_add(ct, 0)
        st.wait()
        plsc.subcore_barrier()
        pltpu.sync_copy(acc.at[stripe], out_hbm.at[cid, stripe])

    return k(pay_flat, dst3, zeros2d)


def _branch(b, nfeats, x0, x1, x2, efeats, ei, dst3, zeros2d,
            We1, be1, We2, be2, Wx, bx, Wh1, bh1, Wh2, bh2):
    h = nfeats
    pay = efeats
    for l in range(5):
        i = b * 5 + l
        ws = We1[i, 0:F, :]
        wd = We1[i, F:2 * F, :]
        we = We1[i, 2 * F:2 * F + De, :]
        wd2 = We1[i, 2 * F + De:2 * F + De + 1, :]
        ps, pdn = _proj(h, ws, wd)
        ga, gb, r0, r1, r2 = _gather_stage(ps, pdn, x0, x1, x2, ei)
        pay = _edge_dense(l == 0, ga, gb, pay, r0, r1, r2,
                          we, wd2, be1[i][None, :], We2[i], be2[i][None, :],
                          Wx[i], bx[i][None, :])
        acc2 = _scatter_stage(pay, dst3, zeros2d)
        h, x0, x1, x2 = _node_update(h, x0, x1, x2, acc2,
                                     Wh1[i, 0:F, :], Wh1[i, F:F + De, :],
                                     bh1[i][None, :], Wh2[i], bh2[i][None, :])
        x0 = x0.reshape(N)
        x1 = x1.reshape(N)
        x2 = x2.reshape(N)
    return h


def kernel(nfeats, coordinates, efeats, edge_index, node_graph_ids,
           We1, be1, We2, be2, Wx, bx, Wh1, bh1, Wh2, bh2,
           Wm, bm, Wd, bd, Wq, bq, Wo, bo):
    dst3 = edge_index[1].reshape(NWORK, NCH, CH)
    ei4 = edge_index.reshape(2, NWORK, NCH, CH)
    x0 = coordinates[:, 0]
    x1 = coordinates[:, 1]
    x2 = coordinates[:, 2]
    zeros2d = jnp.zeros((NP, PW), jnp.float32)
    gid3 = node_graph_ids.reshape(N // NB, 1, NB)
    args = (nfeats, x0, x1, x2, efeats, ei4, dst3, zeros2d,
            We1, be1, We2, be2, Wx, bx, Wh1, bh1, Wh2, bh2)
    h_mon = _branch(0, *args)
    h_dip = _branch(1, *args)
    h_quad = _branch(2, *args)
    h_oct = _branch(3, *args)
    pm_raw, sums, cnt = _mono_readout(h_mon, nfeats, gid3, Wm, bm[None, :])
    fv = _fv_finalize(sums, cnt)
    return _final_readout(pm_raw, gid3, fv, h_dip, h_quad, h_oct,
                          Wd, bd[None, :], Wq, bq[None, :], Wo, bo[None, :])


# speedup vs baseline: 1.0039x; 1.0039x over previous
import functools

import jax
import jax.numpy as jnp
from jax import lax
from jax.experimental import pallas as pl
from jax.experimental.pallas import tpu as pltpu
from jax.experimental.pallas import tpu_sc as plsc

N = 10000
E = 320000
F = 128
De = 16
H = 128
G = 100

NB = 2000
EB = 2560
PW = 32

NWORK = 32
EPW = E // NWORK
CH = 80
NP = 10240
NSTR = NP // 16

_INTERPRET = False


def _silu(v):
    return v * jax.nn.sigmoid(v)


def _wspec(shape):
    nd = len(shape)
    return pl.BlockSpec(shape, lambda *_, **__: (0,) * nd)


def _proj_body(h_ref, ws_ref, wd_ref, ps_ref, pd_ref):
    h = h_ref[...]
    ps_ref[...] = jnp.dot(h, ws_ref[...], preferred_element_type=jnp.float32)
    pd_ref[...] = jnp.dot(h, wd_ref[...], preferred_element_type=jnp.float32)


def _proj(h, ws, wd):
    grid = (N // NB,)
    return pl.pallas_call(
        _proj_body,
        grid=grid,
        in_specs=[pl.BlockSpec((NB, F), lambda i: (i, 0)), _wspec((F, H)), _wspec((F, H))],
        out_specs=[pl.BlockSpec((NB, H), lambda i: (i, 0))] * 2,
        out_shape=[jax.ShapeDtypeStruct((N, H), jnp.float32)] * 2,
        interpret=_INTERPRET,
    )(h, ws, wd)


def _edge_body(first, ga_ref, gb_ref, e_ref, r0_ref, r1_ref, r2_ref,
               we_ref, wd2_ref, be1_ref, we2_ref, be2_ref, wx_ref, bx_ref,
               pay_ref):
    rel_ref = (r0_ref, r1_ref, r2_ref)
    if first:
        e = e_ref[...]
    else:
        e = e_ref[:, 0:De]
    i = pl.program_id(0)
    sl = pl.ds(i * EB, EB)
    r0c = jnp.reshape(rel_ref[0][sl], (EB, 1))
    r1c = jnp.reshape(rel_ref[1][sl], (EB, 1))
    r2c = jnp.reshape(rel_ref[2][sl], (EB, 1))
    d2 = r0c * r0c + r1c * r1c + r2c * r2c
    mpre = (ga_ref[...] + gb_ref[...]
            + jnp.dot(e, we_ref[...], preferred_element_type=jnp.float32)
            + d2 * wd2_ref[...] + be1_ref[...])
    m = _silu(mpre)
    e_new = _silu(jnp.dot(m, we2_ref[...], preferred_element_type=jnp.float32)
                  + be2_ref[...])
    w = jnp.tanh(jnp.dot(e_new, wx_ref[...], preferred_element_type=jnp.float32)
                 + bx_ref[...])
    relw = jnp.concatenate([r0c, r1c, r2c], axis=1) * w
    ones = jnp.ones((EB, 1), jnp.float32)
    zeros = jnp.zeros((EB, PW - De - 4), jnp.float32)
    pay_ref[...] = jnp.concatenate([e_new, relw, ones, zeros], axis=1)


def _edge_dense(first, ga, gb, e, r0, r1, r2, we, wd2, be1, we2, be2, wx, bx):
    grid = (E // EB,)
    e_spec = (pl.BlockSpec((EB, De), lambda i: (i, 0)) if first
              else pl.BlockSpec((EB, PW), lambda i: (i, 0)))
    return pl.pallas_call(
        functools.partial(_edge_body, first),
        grid=grid,
        in_specs=[
            pl.BlockSpec((EB, H), lambda i: (i, 0)),
            pl.BlockSpec((EB, H), lambda i: (i, 0)),
            e_spec,
            _wspec((E,)), _wspec((E,)), _wspec((E,)),
            _wspec((De, H)), _wspec((1, H)), _wspec((1, H)),
            _wspec((H, De)), _wspec((1, De)), _wspec((De, 1)), _wspec((1, 1)),
        ],
        out_specs=pl.BlockSpec((EB, PW), lambda i: (i, 0)),
        out_shape=jax.ShapeDtypeStruct((E, PW), jnp.float32),
        interpret=_INTERPRET,
    )(ga, gb, e, r0, r1, r2, we, wd2, be1, we2, be2, wx, bx)


def _node_body(h_ref, x0_ref, x1_ref, x2_ref, acc_ref, wh1h_ref, wh1a_ref,
               bh1_ref, wh2_ref, bh2_ref, hn_ref, xn0_ref, xn1_ref, xn2_ref):
    acc = acc_ref[0] + acc_ref[1]
    h = h_ref[...]
    inv = 1.0 / jnp.maximum(acc[:, 19:20], 1.0)
    agg = acc[:, 0:16] * inv
    dx = acc[:, 16:19] * inv
    for xr, xnr, c in ((x0_ref, xn0_ref, 0), (x1_ref, xn1_ref, 1),
                       (x2_ref, xn2_ref, 2)):
        xc = jnp.reshape(xr[...], (NB, 1))
        xnr[...] = jnp.reshape(xc + dx[:, c:c + 1], (1, 1, NB))
    hp = _silu(jnp.dot(h, wh1h_ref[...], preferred_element_type=jnp.float32)
               + jnp.dot(agg, wh1a_ref[...], preferred_element_type=jnp.float32)
               + bh1_ref[...])
    hn_ref[...] = h + jnp.dot(hp, wh2_ref[...], preferred_element_type=jnp.float32) + bh2_ref[...]


def _node_update(h, x0, x1, x2, acc2, wh1h, wh1a, bh1, wh2, bh2):
    grid = (N // NB,)
    return pl.pallas_call(
        _node_body,
        grid=grid,
        in_specs=[
            pl.BlockSpec((NB, F), lambda i: (i, 0)),
            pl.BlockSpec((1, 1, NB), lambda i: (i, 0, 0)),
            pl.BlockSpec((1, 1, NB), lambda i: (i, 0, 0)),
            pl.BlockSpec((1, 1, NB), lambda i: (i, 0, 0)),
            pl.BlockSpec((2, NB, PW), lambda i: (0, i, 0)),
            _wspec((F, H)), _wspec((De, H)), _wspec((1, H)),
            _wspec((H, F)), _wspec((1, F)),
        ],
        out_specs=[pl.BlockSpec((NB, F), lambda i: (i, 0)),
                   pl.BlockSpec((1, 1, NB), lambda i: (i, 0, 0)),
                   pl.BlockSpec((1, 1, NB), lambda i: (i, 0, 0)),
                   pl.BlockSpec((1, 1, NB), lambda i: (i, 0, 0))],
        out_shape=[jax.ShapeDtypeStruct((N, F), jnp.float32),
                   jax.ShapeDtypeStruct((N // NB, 1, NB), jnp.float32),
                   jax.ShapeDtypeStruct((N // NB, 1, NB), jnp.float32),
                   jax.ShapeDtypeStruct((N // NB, 1, NB), jnp.float32)],
        interpret=_INTERPRET,
    )(h, x0.reshape(N // NB, 1, NB), x1.reshape(N // NB, 1, NB),
      x2.reshape(N // NB, 1, NB), acc2, wh1h, wh1a, bh1, wh2, bh2)


def _mono_body(h_ref, nf_ref, gid_ref, wm_ref, bm_ref, pm_ref, sums_ref, cnt_ref):
    i = pl.program_id(0)
    h = h_ref[...]
    pm = jnp.dot(h, wm_ref[...], preferred_element_type=jnp.float32) + bm_ref[...]
    mask = nf_ref[:, 0:1] == 1.0
    pm = jnp.where(mask, jnp.abs(pm), pm)
    pm_ref[...] = pm
    gid = gid_ref[0, 0, :]
    oh = (gid[:, None] == jax.lax.broadcasted_iota(jnp.int32, (1, 128), 1)
          ).astype(jnp.float32)
    psum = jnp.dot(oh.T, pm, preferred_element_type=jnp.float32)
    pcnt = jnp.sum(oh, axis=0)[:, None]

    @pl.when(i == 0)
    def _():
        sums_ref[...] = psum
        cnt_ref[...] = pcnt

    @pl.when(i != 0)
    def _():
        sums_ref[...] += psum
        cnt_ref[...] += pcnt


def _mono_readout(h_mon, nfeats, gid3, wm, bm):
    grid = (N // NB,)
    return pl.pallas_call(
        _mono_body,
        grid=grid,
        in_specs=[
            pl.BlockSpec((NB, F), lambda i: (i, 0)),
            pl.BlockSpec((NB, F), lambda i: (i, 0)),
            pl.BlockSpec((1, 1, NB), lambda i: (i, 0, 0)),
            _wspec((F, 1)), _wspec((1, 1)),
        ],
        out_specs=[pl.BlockSpec((NB, 1), lambda i: (i, 0)),
                   _wspec((128, 1)), _wspec((128, 1))],
        out_shape=[jax.ShapeDtypeStruct((N, 1), jnp.float32),
                   jax.ShapeDtypeStruct((128, 1), jnp.float32),
                   jax.ShapeDtypeStruct((128, 1), jnp.float32)],
        interpret=_INTERPRET,
    )(h_mon, nfeats, gid3, wm, bm)


def _fv_body(sums_ref, cnt_ref, fv_ref):
    s = sums_ref[...]
    fv = s / jnp.maximum(cnt_ref[...], 1.0)
    fv_ref[...] = jnp.where(jnp.abs(s) < 0.01, 0.0, fv)


def _fv_finalize(sums, cnt):
    return pl.pallas_call(
        _fv_body,
        in_specs=[_wspec((128, 1)), _wspec((128, 1))],
        out_specs=_wspec((128, 1)),
        out_shape=jax.ShapeDtypeStruct((128, 1), jnp.float32),
        interpret=_INTERPRET,
    )(sums, cnt)


def _final_body(pm_ref, gid_ref, fv_ref, hd_ref, hq_ref, ho_ref,
                wd_ref, bd_ref, wq_ref, bq_ref, wo_ref, bo_ref, out_ref):
    gid = gid_ref[0, 0, :]
    oh = (gid[:, None] == jax.lax.broadcasted_iota(jnp.int32, (1, 128), 1)
          ).astype(jnp.float32)
    pm = pm_ref[...] - jnp.dot(oh, fv_ref[...], preferred_element_type=jnp.float32)
    pd = jnp.dot(hd_ref[...], wd_ref[...], preferred_element_type=jnp.float32) + bd_ref[...]
    pq = jnp.dot(hq_ref[...], wq_ref[...], preferred_element_type=jnp.float32) + bq_ref[...]
    mt = (pq[:, 0:1] + pq[:, 3:4] + pq[:, 5:6]) / 3.0
    c6 = jax.lax.broadcasted_iota(jnp.int32, (1, 6), 1)
    qmask = ((c6 == 0) | (c6 == 3) | (c6 == 5)).astype(jnp.float32)
    pq = pq - mt * qmask
    po = jnp.dot(ho_ref[...], wo_ref[...], preferred_element_type=jnp.float32) + bo_ref[...]
    m0 = (po[:, 0:1] + po[:, 3:4] + po[:, 5:6]) / 3.0
    m1 = (po[:, 6:7] + po[:, 1:2] + po[:, 8:9]) / 3.0
    m2 = (po[:, 9:10] + po[:, 2:3] + po[:, 7:8]) / 3.0
    c10 = jax.lax.broadcasted_iota(jnp.int32, (1, 10), 1)
    g0 = ((c10 == 0) | (c10 == 3) | (c10 == 5)).astype(jnp.float32)
    g1 = ((c10 == 6) | (c10 == 1) | (c10 == 8)).astype(jnp.float32)
    g2 = ((c10 == 9) | (c10 == 2) | (c10 == 7)).astype(jnp.float32)
    po = po - m0 * g0 - m1 * g1 - m2 * g2
    out_ref[...] = jnp.concatenate([pm, pd, pq, po], axis=1)


def _final_readout(pm_raw, gid3, fv, h_dip, h_quad, h_oct, wd, bd, wq, bq, wo, bo):
    grid = (N // NB,)
    return pl.pallas_call(
        _final_body,
        grid=grid,
        in_specs=[
            pl.BlockSpec((NB, 1), lambda i: (i, 0)),
            pl.BlockSpec((1, 1, NB), lambda i: (i, 0, 0)),
            _wspec((128, 1)),
            pl.BlockSpec((NB, F), lambda i: (i, 0)),
            pl.BlockSpec((NB, F), lambda i: (i, 0)),
            pl.BlockSpec((NB, F), lambda i: (i, 0)),
            _wspec((F, 3)), _wspec((1, 3)),
            _wspec((F, 6)), _wspec((1, 6)),
            _wspec((F, 10)), _wspec((1, 10)),
        ],
        out_specs=pl.BlockSpec((NB, 20), lambda i: (i, 0)),
        out_shape=jax.ShapeDtypeStruct((N, 20), jnp.float32),
        interpret=_INTERPRET,
    )(pm_raw, gid3, fv, h_dip, h_quad, h_oct, wd, bd, wq, bq, wo, bo)


NCH = EPW // CH
NPAIR = NCH // 2


def _gather_stage(ps, pd, x0, x1, x2, ei):
    mesh = plsc.VectorSubcoreMesh(core_axis_name="c", subcore_axis_name="s")

    @functools.partial(
        pl.kernel, mesh=mesh,
        out_type=[jax.ShapeDtypeStruct((E, H), jnp.float32),
                  jax.ShapeDtypeStruct((E, H), jnp.float32),
                  jax.ShapeDtypeStruct((E,), jnp.float32),
                  jax.ShapeDtypeStruct((E,), jnp.float32),
                  jax.ShapeDtypeStruct((E,), jnp.float32)],
        scratch_types=[
            pltpu.VMEM((2, NCH, CH), jnp.int32),
            pltpu.VMEM((2, CH, H), jnp.float32),
            pltpu.VMEM((2, CH, H), jnp.float32),
            pltpu.VMEM((2, 6, CH), jnp.float32),
            pltpu.VMEM((2, 3, CH), jnp.float32),
            pltpu.SemaphoreType.DMA,
            pltpu.SemaphoreType.DMA,
            pltpu.SemaphoreType.DMA,
            pltpu.SemaphoreType.DMA,
            pltpu.SemaphoreType.DMA,
            pltpu.SemaphoreType.DMA,
        ],
    )
    def k(ps_hbm, pd_hbm, x0_hbm, x1_hbm, x2_hbm, ei_hbm,
          ga_hbm, gb_hbm, r0_hbm, r1_hbm, r2_hbm,
          eslab, gabuf, gbbuf, xbuf, rbuf, semI, semA, semB, semC, semW, semR):
        wid = lax.axis_index("s") * 2 + lax.axis_index("c")
        xh = (x0_hbm, x1_hbm, x2_hbm)
        rh = (r0_hbm, r1_hbm, r2_hbm)
        pltpu.async_copy(ei_hbm.at[:, wid], eslab, semI).wait()

        def fire_gathers(ci, b):
            src_r = eslab.at[0, ci]
            dst_r = eslab.at[1, ci]
            cpa = pltpu.async_copy(ps_hbm.at[src_r], gabuf.at[b], semA)
            cpb = pltpu.async_copy(pd_hbm.at[dst_r], gbbuf.at[b], semB)
            cps = []
            for d in range(3):
                cps.append(pltpu.async_copy(xh[d].at[src_r], xbuf.at[b, d], semC))
                cps.append(pltpu.async_copy(xh[d].at[dst_r], xbuf.at[b, d + 3], semC))
            return cpa, cpb, cps

        def drain_chunk(ci, b, cpa, cpb, cps):
            base = pl.multiple_of(wid * EPW + ci * CH, 8)
            for cp in cps:
                cp.wait()
            for d in range(3):
                for j in range(CH // 16):
                    sl = pl.ds(j * 16, 16)
                    rbuf[b, d, sl] = xbuf[b, d, sl] - xbuf[b, d + 3, sl]
            wr = [pltpu.async_copy(rbuf.at[b, d], rh[d].at[pl.ds(base, CH)],
                                   semR) for d in range(3)]
            cpa.wait()
            cpb.wait()
            wa = pltpu.async_copy(gabuf.at[b], ga_hbm.at[pl.ds(base, CH)], semW)
            wb = pltpu.async_copy(gbbuf.at[b], gb_hbm.at[pl.ds(base, CH)], semW)
            return wr + [wa, wb]

        def pair_body(kk, carry):
            c0 = kk * 2
            c1 = c0 + 1
            g0 = fire_gathers(c0, 0)
            g1 = fire_gathers(c1, 1)
            w0 = drain_chunk(c0, 0, *g0)
            w1 = drain_chunk(c1, 1, *g1)
            for w in w0 + w1:
                w.wait()
            return carry

        lax.fori_loop(0, NPAIR, pair_body, 0)
        ct = NCH - 1
        gt = fire_gathers(ct, 0)
        wt = drain_chunk(ct, 0, *gt)
        for w in wt:
            w.wait()

    return k(ps, pd, x0, x1, x2, ei)


def _scatter_stage(pay_flat, dst3, zeros2d):
    mesh = plsc.VectorSubcoreMesh(core_axis_name="c", subcore_axis_name="s")

    @functools.partial(
        pl.kernel, mesh=mesh,
        out_type=jax.ShapeDtypeStruct((2, NP, PW), jnp.float32),
        scratch_types=[
            pltpu.VMEM((NCH, CH), jnp.int32),
            pltpu.VMEM((2, CH, PW), jnp.float32),
            pltpu.VMEM_SHARED((NP, PW), jnp.float32),
            pltpu.SemaphoreType.DMA,
            pltpu.SemaphoreType.DMA,
            pltpu.SemaphoreType.DMA,
        ],
    )
    def k(pay_hbm, dst_hbm, z_hbm, out_hbm, dslab, payv, acc, semD, semP, semS):
        cid = lax.axis_index("c")
        sid = lax.axis_index("s")
        wid = sid * 2 + cid
        stripe = pl.ds(sid * NSTR, NSTR)
        cpd = pltpu.async_copy(dst_hbm.at[wid], dslab, semD)
        pltpu.sync_copy(z_hbm.at[stripe], acc.at[stripe])
        cpd.wait()
        plsc.subcore_barrier()

        def fire_pay(ci, b):
            base = pl.multiple_of(wid * EPW + ci * CH, 8)
            return pltpu.async_copy(pay_hbm.at[pl.ds(base, CH)], payv.at[b],
                                    semP)

        def fire_add(ci, b):
            return pltpu.async_copy(payv.at[b], acc.at[dslab.at[ci]], semS)

        def pair_body(kk, carry):
            c0 = kk * 2
            c1 = c0 + 1
            p0 = fire_pay(c0, 0)
            p1 = fire_pay(c1, 1)
            p0.wait()
            s0 = fire_add(c0, 0)
            p1.wait()
            s1 = fire_add(c1, 1)
            s0.wait()
            s1.wait()
            return carry

        lax.fori_loop(0, NPAIR, pair_body, 0)
        ct = NCH - 1
        pt = fire_pay(ct, 0)
        pt.wait()
        st = fire_add(ct, 0)
        st.wait()
        plsc.subcore_barrier()
        pltpu.sync_copy(acc.at[stripe], out_hbm.at[cid, stripe])

    return k(pay_flat, dst3, zeros2d)


def _branch(b, nfeats, x0, x1, x2, efeats, ei, dst3, zeros2d,
            We1, be1, We2, be2, Wx, bx, Wh1, bh1, Wh2, bh2):
    h = nfeats
    pay = efeats
    for l in range(5):
        i = b * 5 + l
        ws = We1[i, 0:F, :]
        wd = We1[i, F:2 * F, :]
        we = We1[i, 2 * F:2 * F + De, :]
        wd2 = We1[i, 2 * F + De:2 * F + De + 1, :]
        ps, pdn = _proj(h, ws, wd)
        ga, gb, r0, r1, r2 = _gather_stage(ps, pdn, x0, x1, x2, ei)
        pay = _edge_dense(l == 0, ga, gb, pay, r0, r1, r2,
                          we, wd2, be1[i][None, :], We2[i], be2[i][None, :],
                          Wx[i], bx[i][None, :])
        acc2 = _scatter_stage(pay, dst3, zeros2d)
        h, x0, x1, x2 = _node_update(h, x0, x1, x2, acc2,
                                     Wh1[i, 0:F, :], Wh1[i, F:F + De, :],
                                     bh1[i][None, :], Wh2[i], bh2[i][None, :])
        x0 = x0.reshape(N)
        x1 = x1.reshape(N)
        x2 = x2.reshape(N)
    return h


def kernel(nfeats, coordinates, efeats, edge_index, node_graph_ids,
           We1, be1, We2, be2, Wx, bx, Wh1, bh1, Wh2, bh2,
           Wm, bm, Wd, bd, Wq, bq, Wo, bo):
    dst3 = edge_index[1].reshape(NWORK, NCH, CH)
    ei4 = edge_index.reshape(2, NWORK, NCH, CH)
    x0 = coordinates[:, 0]
    x1 = coordinates[:, 1]
    x2 = coordinates[:, 2]
    zeros2d = jnp.zeros((NP, PW), jnp.float32)
    gid3 = node_graph_ids.reshape(N // NB, 1, NB)
    args = (nfeats, x0, x1, x2, efeats, ei4, dst3, zeros2d,
            We1, be1, We2, be2, Wx, bx, Wh1, bh1, Wh2, bh2)
    h_mon = _branch(0, *args)
    h_dip = _branch(1, *args)
    h_quad = _branch(2, *args)
    h_oct = _branch(3, *args)
    pm_raw, sums, cnt = _mono_readout(h_mon, nfeats, gid3, Wm, bm[None, :])
    fv = _fv_finalize(sums, cnt)
    return _final_readout(pm_raw, gid3, fv, h_dip, h_quad, h_oct,
                          Wd, bd[None, :], Wq, bq[None, :], Wo, bo[None, :])
